# baseline (speedup 1.0000x reference)
import functools

import jax
import jax.numpy as jnp
from jax.experimental import pallas as pl
from jax.experimental.pallas import tpu as pltpu


def _fused_kernel(x_ref, w1t_ref, w2t_ref, bcv_ref, wfc_ref, bfc_ref,
                  xout_ref, y_ref, x1_ref, x3_ref,
                  *, hid, out_dim, inv_hw, inv_ohw):
    x48 = x_ref[0]
    xout_ref[0] = x48
    C = x48.shape[0]
    hw = x48.shape[1] * x48.shape[2]

    xb = x48.astype(jnp.bfloat16).reshape(C, hw)

    b1 = bcv_ref[0:hid, :]
    b2 = bcv_ref[hid:hid + out_dim, :]

    h = jnp.dot(w1t_ref[...], xb, preferred_element_type=jnp.float32) + b1
    h = jnp.maximum(h, 0.0)
    y = jnp.dot(w2t_ref[...], h.astype(jnp.bfloat16),
                preferred_element_type=jnp.float32) + b2
    y_ref[0] = y

    xs = jnp.sum(x48, axis=1)
    pooled = jnp.sum(xs, axis=1, keepdims=True) * inv_hw
    wfc1 = wfc_ref[0:C, :]
    wfc2t = wfc_ref[C:, :]
    hfc = jax.lax.dot_general(
        pooled, wfc1, (((0,), (0,)), ((), ())),
        preferred_element_type=jnp.float32)
    hfc = jnp.maximum(hfc + bfc_ref[:, 0:hid], 0.0)
    x1 = jax.lax.dot_general(
        hfc, wfc2t, (((1,), (1,)), ((), ())),
        preferred_element_type=jnp.float32)
    x1_ref[0] = x1 + bfc_ref[:, hid:hid + out_dim]
    x3_ref[0] = (jnp.sum(y) * inv_ohw).reshape(1, 1)


def kernel(x, w1_fc, b1_fc, w2_fc, b2_fc, w1_cv, b1_cv, w2_cv, b2_cv):
    B, C, H, W = x.shape
    HW = H * W
    S = HW // 128
    hid = w1_cv.shape[1]
    out_dim = w2_cv.shape[1]

    xv = x.reshape(B, C, S, 128)
    w1t = w1_cv.T.astype(jnp.bfloat16)
    w2t = w2_cv.T.astype(jnp.bfloat16)
    bcv = jnp.concatenate([b1_cv, b2_cv]).reshape(hid + out_dim, 1)
    wfc = jnp.concatenate([w1_fc, w2_fc.T], axis=0)
    bfc = jnp.concatenate([b1_fc, b2_fc]).reshape(1, hid + out_dim)

    body = functools.partial(_fused_kernel,
                             hid=hid, out_dim=out_dim,
                             inv_hw=1.0 / HW,
                             inv_ohw=1.0 / (out_dim * HW))

    xoutv, y, x1o, x3o = pl.pallas_call(
        body,
        grid=(B,),
        in_specs=[
            pl.BlockSpec((1, C, S, 128), lambda b: (b, 0, 0, 0)),
            pl.BlockSpec((hid, C), lambda b: (0, 0)),
            pl.BlockSpec((out_dim, hid), lambda b: (0, 0)),
            pl.BlockSpec((hid + out_dim, 1), lambda b: (0, 0)),
            pl.BlockSpec((C + out_dim, hid), lambda b: (0, 0)),
            pl.BlockSpec((1, hid + out_dim), lambda b: (0, 0)),
        ],
        out_specs=[
            pl.BlockSpec((1, C, S, 128), lambda b: (b, 0, 0, 0)),
            pl.BlockSpec((1, out_dim, HW), lambda b: (b, 0, 0)),
            pl.BlockSpec((1, 1, out_dim), lambda b: (b, 0, 0)),
            pl.BlockSpec((1, 1, 1), lambda b: (b, 0, 0)),
        ],
        out_shape=[
            jax.ShapeDtypeStruct((B, C, S, 128), jnp.float32),
            jax.ShapeDtypeStruct((B, out_dim, HW), jnp.float32),
            jax.ShapeDtypeStruct((B, 1, out_dim), jnp.float32),
            jax.ShapeDtypeStruct((B, 1, 1), jnp.float32),
        ],
        compiler_params=pltpu.CompilerParams(
            dimension_semantics=("parallel",)),
    )(xv, w1t, w2t, bcv, wfc, bfc)

    xout = xoutv.reshape(B, C, H, W)
    x1 = x1o[:, 0, :]
    x3 = x3o[:, :, 0]
    return xout, x1, y, x3
